# Initial kernel scaffold; baseline (speedup 1.0000x reference)
import jax
import jax.numpy as jnp
from jax.experimental import pallas as pl


def kernel(node_emb, edge_index, W1, b1, W2, b2, eps):
    raise NotImplementedError("write your pallas kernel here")


# speedup vs baseline: 1.5571x; 1.5571x over previous
import functools

import jax
import jax.numpy as jnp
from jax import lax
from jax.experimental import pallas as pl
from jax.experimental.pallas import tpu as pltpu
from jax.experimental.pallas import tpu_sc as plsc

N = 10000
E = 320000
D = 128
H = 64

NC = 2
NS = 16
NW = NC * NS
EPW = E // NW
C = 80
NCHUNK = EPW // C
G = C // 16


def _tc_body(ne_ref, w1s_ref, w1d_ref, b1_ref, eps_ref, b2_ref,
             a_ref, b_ref, g_ref):
    ne = ne_ref[...]
    a = jnp.dot(ne, w1s_ref[...], preferred_element_type=jnp.float32)
    a_ref[...] = 0.5 * (a + b1_ref[...])
    b_ref[...] = 0.5 * jnp.dot(ne, w1d_ref[...],
                               preferred_element_type=jnp.float32)
    eps = eps_ref[...]
    g_ref[...] = 0.5 * (jnp.log(eps) - jnp.log(1.0 - eps)) + 0.5 * b2_ref[0, 0]


def _tc_precompute(node_emb, w1s, w1d, b1r, eps2d, b2r):
    return pl.pallas_call(
        _tc_body,
        out_shape=[
            jax.ShapeDtypeStruct((N, H), jnp.float32),
            jax.ShapeDtypeStruct((N, H), jnp.float32),
            jax.ShapeDtypeStruct(eps2d.shape, jnp.float32),
        ],
    )(node_emb, w1s, w1d, b1r, eps2d, b2r)


_mesh = plsc.VectorSubcoreMesh(core_axis_name="c", subcore_axis_name="s")


@functools.partial(
    pl.kernel,
    out_type=jax.ShapeDtypeStruct((E,), jnp.float32),
    mesh=_mesh,
    scratch_types=[
        pltpu.VMEM((C,), jnp.int32),
        pltpu.VMEM((C,), jnp.int32),
        pltpu.VMEM((C, H), jnp.float32),
        pltpu.VMEM((C, H), jnp.float32),
        pltpu.VMEM((C,), jnp.float32),
        pltpu.VMEM((C,), jnp.float32),
        pltpu.VMEM((H,), jnp.float32),
        pltpu.SemaphoreType.DMA,
        pltpu.SemaphoreType.DMA,
    ],
    compiler_params=pltpu.CompilerParams(
        needs_layout_passes=False, use_tc_tiling_on_sc=False),
)
def _sc_edge_gate(a_hbm, b_hbm, src_hbm, dst_hbm, g_hbm, w2_hbm, out_hbm,
                  sidx, didx, a_rows, b_rows, gv, outv, w2v, sem1, sem2):
    wid = lax.axis_index("s") * NC + lax.axis_index("c")
    base_w = wid * EPW
    pltpu.sync_copy(w2_hbm, w2v)
    w2r = [w2v[pl.ds(k * 16, 16)] for k in range(H // 16)]
    iot = lax.iota(jnp.int32, 16)

    @pl.loop(0, NCHUNK)
    def _chunk(i):
        base = base_w + i * C
        pltpu.sync_copy(src_hbm.at[pl.ds(base, C)], sidx)
        pltpu.sync_copy(dst_hbm.at[pl.ds(base, C)], didx)
        pltpu.sync_copy(g_hbm.at[pl.ds(base, C)], gv)
        cp1 = pltpu.async_copy(a_hbm.at[sidx], a_rows, sem1)
        cp2 = pltpu.async_copy(b_hbm.at[didx], b_rows, sem2)
        cp1.wait()
        cp2.wait()

        @pl.loop(0, G)
        def _group(g):
            rows = g * 16 + iot
            acc = jnp.zeros((16,), jnp.float32)
            for dd in range(H):
                cols = jnp.full((16,), dd, jnp.int32)
                av = plsc.load_gather(a_rows, [rows, cols])
                bv = plsc.load_gather(b_rows, [rows, cols])
                acc = acc + jnp.maximum(av + bv, 0.0) * w2r[dd // 16][dd % 16]
            z = acc + gv[pl.ds(g * 16, 16)]
            m = 1.0 / (1.0 + jnp.exp(-z))
            outv[pl.ds(g * 16, 16)] = jnp.maximum(m, 0.0)

        pltpu.sync_copy(outv, out_hbm.at[pl.ds(base, C)])


def kernel(node_emb, edge_index, W1, b1, W2, b2, eps):
    w1s = W1[:D]
    w1d = W1[D:]
    b1r = b1.reshape(1, H)
    b2r = b2.reshape(1, 1)
    eps2d = eps.reshape(E // D, D)
    a_tab, b_tab, g2d = _tc_precompute(node_emb, w1s, w1d, b1r, eps2d, b2r)
    g_edge = g2d.reshape(E)
    src = edge_index[0]
    dst = edge_index[1]
    w2f = W2.reshape(H)
    return _sc_edge_gate(a_tab, b_tab, src, dst, g_edge, w2f)


# speedup vs baseline: 2.3251x; 1.4932x over previous
import functools

import jax
import jax.numpy as jnp
from jax import lax
from jax.experimental import pallas as pl
from jax.experimental.pallas import tpu as pltpu
from jax.experimental.pallas import tpu_sc as plsc

N = 10000
E = 320000
D = 128
H = 64

NC = 2
NS = 16
NW = NC * NS
EPW = E // NW
C2 = 400
SUB = 80
NSUB = C2 // SUB
NCH = EPW // C2
NCHT = E // C2
GR = C2 // 16


def _tc_body(ne_ref, w1s_ref, w1d_ref, b1_ref, eps_ref, b2_ref,
             a_ref, b_ref, g_ref):
    ne = ne_ref[...]
    a = jnp.dot(ne, w1s_ref[...], preferred_element_type=jnp.float32)
    a_ref[...] = 0.5 * (a + b1_ref[...])
    b_ref[...] = 0.5 * jnp.dot(ne, w1d_ref[...],
                               preferred_element_type=jnp.float32)
    eps = eps_ref[...]
    g_ref[...] = 0.5 * (jnp.log(eps) - jnp.log(1.0 - eps)) + 0.5 * b2_ref[0, 0]


def _tc_precompute(node_emb, w1s, w1d, b1r, eps2d, b2r):
    return pl.pallas_call(
        _tc_body,
        out_shape=[
            jax.ShapeDtypeStruct((N, H), jnp.float32),
            jax.ShapeDtypeStruct((N, H), jnp.float32),
            jax.ShapeDtypeStruct(eps2d.shape, jnp.float32),
        ],
    )(node_emb, w1s, w1d, b1r, eps2d, b2r)


_mesh = plsc.VectorSubcoreMesh(core_axis_name="c", subcore_axis_name="s")


@functools.partial(
    pl.kernel,
    out_type=jax.ShapeDtypeStruct((E,), jnp.float32),
    mesh=_mesh,
    scratch_types=[
        pltpu.VMEM((2, C2), jnp.int32),
        pltpu.VMEM((2, C2), jnp.int32),
        pltpu.VMEM((C2, H), jnp.float32),
        pltpu.VMEM((C2, H), jnp.float32),
        pltpu.VMEM((C2, H), jnp.float32),
        pltpu.VMEM((C2, H), jnp.float32),
        pltpu.VMEM((C2,), jnp.float32),
        pltpu.VMEM((C2,), jnp.float32),
        pltpu.VMEM((C2,), jnp.float32),
        pltpu.VMEM((C2,), jnp.float32),
        pltpu.VMEM((H,), jnp.float32),
        pltpu.SemaphoreType.DMA,
        pltpu.SemaphoreType.DMA,
        pltpu.SemaphoreType.DMA,
        pltpu.SemaphoreType.DMA,
        pltpu.SemaphoreType.DMA,
        pltpu.SemaphoreType.DMA,
    ],
    compiler_params=pltpu.CompilerParams(
        needs_layout_passes=False, use_tc_tiling_on_sc=False),
)
def _sc_edge_gate(a_hbm, b_hbm, meta_hbm, g_hbm, w2_hbm, out_hbm,
                  meta0, meta1, ar0, ar1, br0, br1, gv0, gv1, ov0, ov1, w2v,
                  semm0, semm1, semg0, semg1, semo0, semo1):
    metas = (meta0, meta1)
    ars = (ar0, ar1)
    brs = (br0, br1)
    gvs = (gv0, gv1)
    ovs = (ov0, ov1)
    semms = (semm0, semm1)
    semgs = (semg0, semg1)
    semos = (semo0, semo1)

    wid = lax.axis_index("s") * NC + lax.axis_index("c")
    cbase = wid * NCH
    ebase = wid * EPW

    pltpu.sync_copy(w2_hbm, w2v)
    w2r = [w2v[pl.ds(k * 16, 16)] for k in range(H // 16)]
    iot = lax.iota(jnp.int32, 16)

    pltpu.async_copy(meta_hbm.at[cbase], metas[0], semms[0])
    pltpu.async_copy(meta_hbm.at[cbase + 1], metas[1], semms[1])

    @pl.loop(0, NCH + 1, step=2)
    def _outer(i):
        for b in range(2):
            bp = 1 - b
            ci = i + b

            @pl.when(ci < NCH)
            def _fire():
                pltpu.make_async_copy(
                    meta_hbm.at[cbase + ci], metas[b], semms[b]).wait()
                for j in range(NSUB):
                    sl = pl.ds(j * SUB, SUB)
                    pltpu.async_copy(
                        a_hbm.at[metas[b].at[0, sl]], ars[b].at[sl], semgs[b])
                    pltpu.async_copy(
                        b_hbm.at[metas[b].at[1, sl]], brs[b].at[sl], semgs[b])
                pltpu.async_copy(
                    g_hbm.at[pl.ds(ebase + ci * C2, C2)], gvs[b], semgs[b])

            @pl.when(ci > 0)
            def _consume():
                cp = ci - 1
                for j in range(NSUB):
                    sl = pl.ds(j * SUB, SUB)
                    pltpu.make_async_copy(
                        a_hbm.at[metas[bp].at[0, sl]], ars[bp].at[sl],
                        semgs[bp]).wait()
                    pltpu.make_async_copy(
                        b_hbm.at[metas[bp].at[1, sl]], brs[bp].at[sl],
                        semgs[bp]).wait()
                pltpu.make_async_copy(
                    g_hbm.at[pl.ds(ebase + cp * C2, C2)], gvs[bp],
                    semgs[bp]).wait()

                @pl.when(ci + 1 < NCH)
                def _pref():
                    pltpu.async_copy(
                        meta_hbm.at[cbase + ci + 1], metas[bp], semms[bp])

                @pl.when(ci > 2)
                def _drain_out():
                    pltpu.make_async_copy(
                        g_hbm.at[pl.ds(0, C2)], ovs[bp], semos[bp]).wait()

                @pl.loop(0, GR)
                def _group(g):
                    rows = g * 16 + iot
                    acc = jnp.zeros((16,), jnp.float32)
                    for dd in range(H):
                        cols = jnp.full((16,), dd, jnp.int32)
                        av = plsc.load_gather(ars[bp], [rows, cols])
                        bv = plsc.load_gather(brs[bp], [rows, cols])
                        acc = acc + (jnp.maximum(av + bv, 0.0)
                                     * w2r[dd // 16][dd % 16])
                    z = acc + gvs[bp][pl.ds(g * 16, 16)]
                    m = 1.0 / (1.0 + jnp.exp(-z))
                    ovs[bp][pl.ds(g * 16, 16)] = jnp.maximum(m, 0.0)

                pltpu.async_copy(
                    ovs[bp], out_hbm.at[pl.ds(ebase + cp * C2, C2)], semos[bp])

    pltpu.make_async_copy(g_hbm.at[pl.ds(0, C2)], ovs[0], semos[0]).wait()
    pltpu.make_async_copy(g_hbm.at[pl.ds(0, C2)], ovs[1], semos[1]).wait()


def kernel(node_emb, edge_index, W1, b1, W2, b2, eps):
    w1s = W1[:D]
    w1d = W1[D:]
    b1r = b1.reshape(1, H)
    b2r = b2.reshape(1, 1)
    eps2d = eps.reshape(E // D, D)
    a_tab, b_tab, g2d = _tc_precompute(node_emb, w1s, w1d, b1r, eps2d, b2r)
    g_edge = g2d.reshape(E)
    meta = jnp.stack(
        [edge_index[0].reshape(NCHT, C2), edge_index[1].reshape(NCHT, C2)],
        axis=1)
    w2f = W2.reshape(H)
    return _sc_edge_gate(a_tab, b_tab, meta, g_edge, w2f)


# speedup vs baseline: 11.0519x; 4.7534x over previous
import functools

import jax
import jax.numpy as jnp
from jax import lax
from jax.experimental import pallas as pl
from jax.experimental.pallas import tpu as pltpu
from jax.experimental.pallas import tpu_sc as plsc

N = 10000
E = 320000
D = 128
H = 64

NC = 2
NS = 16
NW = NC * NS
EPW = E // NW
C2 = 400
SUB = 80
NSUB = C2 // SUB
NCH = EPW // C2
NCHT = E // C2
GR = C2 // 16


def _tc_body(ne_ref, w1s_ref, w1d_ref, b1_ref, eps_ref, b2_ref,
             a_ref, b_ref, g_ref):
    ne = ne_ref[...]
    a = jnp.dot(ne, w1s_ref[...], preferred_element_type=jnp.float32)
    a_ref[...] = 0.5 * (a + b1_ref[...])
    b_ref[...] = 0.5 * jnp.dot(ne, w1d_ref[...],
                               preferred_element_type=jnp.float32)
    eps = eps_ref[...]
    g_ref[...] = 0.5 * (jnp.log(eps) - jnp.log(1.0 - eps)) + 0.5 * b2_ref[0, 0]


def _tc_precompute(node_emb, w1s, w1d, b1r, eps2d, b2r):
    return pl.pallas_call(
        _tc_body,
        out_shape=[
            jax.ShapeDtypeStruct((N, H), jnp.float32),
            jax.ShapeDtypeStruct((N, H), jnp.float32),
            jax.ShapeDtypeStruct(eps2d.shape, jnp.float32),
        ],
    )(node_emb, w1s, w1d, b1r, eps2d, b2r)


_mesh = plsc.VectorSubcoreMesh(core_axis_name="c", subcore_axis_name="s")


@functools.partial(
    pl.kernel,
    out_type=jax.ShapeDtypeStruct((E,), jnp.float32),
    mesh=_mesh,
    scratch_types=[
        pltpu.VMEM((2, C2), jnp.int32),
        pltpu.VMEM((2, C2), jnp.int32),
        pltpu.VMEM((C2, H), jnp.float32),
        pltpu.VMEM((C2, H), jnp.float32),
        pltpu.VMEM((C2, H), jnp.float32),
        pltpu.VMEM((C2, H), jnp.float32),
        pltpu.VMEM((C2,), jnp.float32),
        pltpu.VMEM((C2,), jnp.float32),
        pltpu.VMEM((C2,), jnp.float32),
        pltpu.VMEM((C2,), jnp.float32),
        pltpu.VMEM((H,), jnp.float32),
        pltpu.SemaphoreType.DMA,
        pltpu.SemaphoreType.DMA,
        pltpu.SemaphoreType.DMA,
        pltpu.SemaphoreType.DMA,
        pltpu.SemaphoreType.DMA,
        pltpu.SemaphoreType.DMA,
    ],
    compiler_params=pltpu.CompilerParams(
        needs_layout_passes=False, use_tc_tiling_on_sc=False),
)
def _sc_edge_gate(a_hbm, b_hbm, meta_hbm, g_hbm, w2_hbm, out_hbm,
                  meta0, meta1, ar0, ar1, br0, br1, gv0, gv1, ov0, ov1, w2v,
                  semm0, semm1, semg0, semg1, semo0, semo1):
    metas = (meta0, meta1)
    ars = (ar0, ar1)
    brs = (br0, br1)
    gvs = (gv0, gv1)
    ovs = (ov0, ov1)
    semms = (semm0, semm1)
    semgs = (semg0, semg1)
    semos = (semo0, semo1)

    wid = lax.axis_index("s") * NC + lax.axis_index("c")
    cbase = wid * NCH
    ebase = wid * EPW

    pltpu.sync_copy(w2_hbm, w2v)
    w2r = [w2v[pl.ds(k * 16, 16)] for k in range(H // 16)]
    iot = lax.iota(jnp.int32, 16)

    pltpu.async_copy(meta_hbm.at[cbase], metas[0], semms[0])
    pltpu.async_copy(meta_hbm.at[cbase + 1], metas[1], semms[1])

    @pl.loop(0, NCH + 1, step=2)
    def _outer(i):
        for b in range(2):
            bp = 1 - b
            ci = i + b

            @pl.when(ci < NCH)
            def _fire():
                pltpu.make_async_copy(
                    meta_hbm.at[cbase + ci], metas[b], semms[b]).wait()
                for j in range(NSUB):
                    sl = pl.ds(j * SUB, SUB)
                    pltpu.async_copy(
                        a_hbm.at[metas[b].at[0, sl]], ars[b].at[sl], semgs[b])
                    pltpu.async_copy(
                        b_hbm.at[metas[b].at[1, sl]], brs[b].at[sl], semgs[b])
                pltpu.async_copy(
                    g_hbm.at[pl.ds(ebase + ci * C2, C2)], gvs[b], semgs[b])

            @pl.when(ci > 0)
            def _consume():
                cp = ci - 1
                for j in range(NSUB):
                    sl = pl.ds(j * SUB, SUB)
                    pltpu.make_async_copy(
                        a_hbm.at[metas[bp].at[0, sl]], ars[bp].at[sl],
                        semgs[bp]).wait()
                    pltpu.make_async_copy(
                        b_hbm.at[metas[bp].at[1, sl]], brs[bp].at[sl],
                        semgs[bp]).wait()
                pltpu.make_async_copy(
                    g_hbm.at[pl.ds(ebase + cp * C2, C2)], gvs[bp],
                    semgs[bp]).wait()

                @pl.when(ci + 1 < NCH)
                def _pref():
                    pltpu.async_copy(
                        meta_hbm.at[cbase + ci + 1], metas[bp], semms[bp])

                @pl.when(ci > 2)
                def _drain_out():
                    pltpu.make_async_copy(
                        g_hbm.at[pl.ds(0, C2)], ovs[bp], semos[bp]).wait()

                @pl.loop(0, GR)
                def _group(g):
                    base = g * 16
                    parts = []
                    for e in range(16):
                        row = base + e
                        v = None
                        for k in range(H // 16):
                            sk = pl.ds(k * 16, 16)
                            h = jnp.maximum(
                                ars[bp][row, sk] + brs[bp][row, sk], 0.0)
                            q = h * w2r[k]
                            v = q if v is None else v + q
                        parts.append(jnp.where(iot == e, jnp.sum(v), 0.0))
                    while len(parts) > 1:
                        parts = [parts[i] + parts[i + 1]
                                 for i in range(0, len(parts), 2)]
                    z = parts[0] + gvs[bp][pl.ds(base, 16)]
                    m = 1.0 / (1.0 + jnp.exp(-z))
                    ovs[bp][pl.ds(base, 16)] = jnp.maximum(m, 0.0)

                pltpu.async_copy(
                    ovs[bp], out_hbm.at[pl.ds(ebase + cp * C2, C2)], semos[bp])

    pltpu.make_async_copy(g_hbm.at[pl.ds(0, C2)], ovs[0], semos[0]).wait()
    pltpu.make_async_copy(g_hbm.at[pl.ds(0, C2)], ovs[1], semos[1]).wait()


def kernel(node_emb, edge_index, W1, b1, W2, b2, eps):
    w1s = W1[:D]
    w1d = W1[D:]
    b1r = b1.reshape(1, H)
    b2r = b2.reshape(1, 1)
    eps2d = eps.reshape(E // D, D)
    a_tab, b_tab, g2d = _tc_precompute(node_emb, w1s, w1d, b1r, eps2d, b2r)
    g_edge = g2d.reshape(E)
    meta = jnp.stack(
        [edge_index[0].reshape(NCHT, C2), edge_index[1].reshape(NCHT, C2)],
        axis=1)
    w2f = W2.reshape(H)
    return _sc_edge_gate(a_tab, b_tab, meta, g_edge, w2f)


# speedup vs baseline: 12.7358x; 1.1524x over previous
import functools

import jax
import jax.numpy as jnp
from jax import lax
from jax.experimental import pallas as pl
from jax.experimental.pallas import tpu as pltpu
from jax.experimental.pallas import tpu_sc as plsc

N = 10000
E = 320000
D = 128
H = 64

NC = 2
NS = 16
NW = NC * NS
EPW = E // NW
C2 = 400
SUB = 80
NSUB = C2 // SUB
NCH = EPW // C2
NCHT = E // C2
GR = C2 // 16


def _tc_body(ne_ref, w1s_ref, w1d_ref, b1_ref, eps_ref, b2_ref,
             a_ref, b_ref, g_ref):
    ne = ne_ref[...]
    a = jnp.dot(ne, w1s_ref[...], preferred_element_type=jnp.float32)
    a_ref[...] = (0.5 * (a + b1_ref[...])).astype(jnp.bfloat16)
    b_ref[...] = (0.5 * jnp.dot(ne, w1d_ref[...],
                                preferred_element_type=jnp.float32)
                  ).astype(jnp.bfloat16)
    eps = eps_ref[...]
    g_ref[...] = 0.5 * (jnp.log(eps) - jnp.log(1.0 - eps)) + 0.5 * b2_ref[0, 0]


def _tc_precompute(node_emb, w1s, w1d, b1r, eps2d, b2r):
    return pl.pallas_call(
        _tc_body,
        out_shape=[
            jax.ShapeDtypeStruct((N, H), jnp.bfloat16),
            jax.ShapeDtypeStruct((N, H), jnp.bfloat16),
            jax.ShapeDtypeStruct(eps2d.shape, jnp.float32),
        ],
    )(node_emb, w1s, w1d, b1r, eps2d, b2r)


_mesh = plsc.VectorSubcoreMesh(core_axis_name="c", subcore_axis_name="s")


@functools.partial(
    pl.kernel,
    out_type=jax.ShapeDtypeStruct((E,), jnp.float32),
    mesh=_mesh,
    scratch_types=[
        pltpu.VMEM((2, C2), jnp.int32),
        pltpu.VMEM((2, C2), jnp.int32),
        pltpu.VMEM((C2, H), jnp.bfloat16),
        pltpu.VMEM((C2, H), jnp.bfloat16),
        pltpu.VMEM((C2, H), jnp.bfloat16),
        pltpu.VMEM((C2, H), jnp.bfloat16),
        pltpu.VMEM((C2,), jnp.float32),
        pltpu.VMEM((C2,), jnp.float32),
        pltpu.VMEM((C2,), jnp.float32),
        pltpu.VMEM((C2,), jnp.float32),
        pltpu.VMEM((H,), jnp.bfloat16),
        pltpu.SemaphoreType.DMA,
        pltpu.SemaphoreType.DMA,
        pltpu.SemaphoreType.DMA,
        pltpu.SemaphoreType.DMA,
        pltpu.SemaphoreType.DMA,
        pltpu.SemaphoreType.DMA,
    ],
    compiler_params=pltpu.CompilerParams(
        needs_layout_passes=False, use_tc_tiling_on_sc=False),
)
def _sc_edge_gate(a_hbm, b_hbm, meta_hbm, g_hbm, w2_hbm, out_hbm,
                  meta0, meta1, ar0, ar1, br0, br1, gv0, gv1, ov0, ov1, w2v,
                  semm0, semm1, semg0, semg1, semo0, semo1):
    metas = (meta0, meta1)
    ars = (ar0, ar1)
    brs = (br0, br1)
    gvs = (gv0, gv1)
    ovs = (ov0, ov1)
    semms = (semm0, semm1)
    semgs = (semg0, semg1)
    semos = (semo0, semo1)

    wid = lax.axis_index("s") * NC + lax.axis_index("c")
    cbase = wid * NCH
    ebase = wid * EPW

    pltpu.sync_copy(w2_hbm, w2v)
    w2r = [w2v[pl.ds(k * 32, 32)] for k in range(H // 32)]
    iot = lax.iota(jnp.int32, 16)

    pltpu.async_copy(meta_hbm.at[cbase], metas[0], semms[0])
    pltpu.async_copy(meta_hbm.at[cbase + 1], metas[1], semms[1])

    @pl.loop(0, NCH + 1, step=2)
    def _outer(i):
        for b in range(2):
            bp = 1 - b
            ci = i + b

            @pl.when(ci < NCH)
            def _fire():
                pltpu.make_async_copy(
                    meta_hbm.at[cbase + ci], metas[b], semms[b]).wait()
                for j in range(NSUB):
                    sl = pl.ds(j * SUB, SUB)
                    pltpu.async_copy(
                        a_hbm.at[metas[b].at[0, sl]], ars[b].at[sl], semgs[b])
                    pltpu.async_copy(
                        b_hbm.at[metas[b].at[1, sl]], brs[b].at[sl], semgs[b])
                pltpu.async_copy(
                    g_hbm.at[pl.ds(ebase + ci * C2, C2)], gvs[b], semgs[b])

            @pl.when(ci > 0)
            def _consume():
                cp = ci - 1
                for j in range(NSUB):
                    sl = pl.ds(j * SUB, SUB)
                    pltpu.make_async_copy(
                        a_hbm.at[metas[bp].at[0, sl]], ars[bp].at[sl],
                        semgs[bp]).wait()
                    pltpu.make_async_copy(
                        b_hbm.at[metas[bp].at[1, sl]], brs[bp].at[sl],
                        semgs[bp]).wait()
                pltpu.make_async_copy(
                    g_hbm.at[pl.ds(ebase + cp * C2, C2)], gvs[bp],
                    semgs[bp]).wait()

                @pl.when(ci + 1 < NCH)
                def _pref():
                    pltpu.async_copy(
                        meta_hbm.at[cbase + ci + 1], metas[bp], semms[bp])

                @pl.when(ci > 2)
                def _drain_out():
                    pltpu.make_async_copy(
                        g_hbm.at[pl.ds(0, C2)], ovs[bp], semos[bp]).wait()

                @pl.loop(0, GR)
                def _group(g):
                    base = g * 16
                    parts = []
                    for e in range(16):
                        row = base + e
                        v = None
                        for k in range(H // 32):
                            sk = pl.ds(k * 32, 32)
                            h = jnp.maximum(
                                ars[bp][row, sk] + brs[bp][row, sk],
                                jnp.bfloat16(0.0))
                            q0, q1 = plsc.unpack(
                                h * w2r[k], format=plsc.PackFormat.INTERLEAVED)
                            q = q0 + q1
                            v = q if v is None else v + q
                        parts.append(jnp.where(iot == e, jnp.sum(v), 0.0))
                    while len(parts) > 1:
                        parts = [parts[i] + parts[i + 1]
                                 for i in range(0, len(parts), 2)]
                    z = parts[0] + gvs[bp][pl.ds(base, 16)]
                    m = 1.0 / (1.0 + jnp.exp(-z))
                    ovs[bp][pl.ds(base, 16)] = jnp.maximum(m, 0.0)

                pltpu.async_copy(
                    ovs[bp], out_hbm.at[pl.ds(ebase + cp * C2, C2)], semos[bp])

    pltpu.make_async_copy(g_hbm.at[pl.ds(0, C2)], ovs[0], semos[0]).wait()
    pltpu.make_async_copy(g_hbm.at[pl.ds(0, C2)], ovs[1], semos[1]).wait()


def kernel(node_emb, edge_index, W1, b1, W2, b2, eps):
    w1s = W1[:D]
    w1d = W1[D:]
    b1r = b1.reshape(1, H)
    b2r = b2.reshape(1, 1)
    eps2d = eps.reshape(E // D, D)
    a_tab, b_tab, g2d = _tc_precompute(node_emb, w1s, w1d, b1r, eps2d, b2r)
    g_edge = g2d.reshape(E)
    meta = jnp.stack(
        [edge_index[0].reshape(NCHT, C2), edge_index[1].reshape(NCHT, C2)],
        axis=1)
    w2f = W2.reshape(H).astype(jnp.bfloat16)
    return _sc_edge_gate(a_tab, b_tab, meta, g_edge, w2f)
